# baseline (speedup 1.0000x reference)
import jax
import jax.numpy as jnp
from jax import lax
from jax.experimental import pallas as pl
from jax.experimental.pallas import tpu as pltpu
from jax.experimental.pallas import tpu_sc as plsc

_B, _C, _E, _P = 32, 64, 64, 256
_L = 16
_NC, _NS = 2, 16
_NW = _NC * _NS
_CPW = _C // _NW
_NSLAB = _B * _CPW


def _gather_slab(in_ref, out_ref, idx_v, cl):
    base = [idx_v[pl.ds(cl * _P + j * _L, _L)] for j in range(_P // _L)]

    @plsc.parallel_loop(0, _E, unroll=4)
    def e_body(e):
        e_vec = jnp.full((_L,), e, dtype=jnp.int32)
        for j in range(_P // _L):
            g = plsc.load_gather(in_ref, [e_vec, base[j]])
            out_ref[e, pl.ds(j * _L, _L)] = g


_DI = 4


def _shuffle_body(x_hbm, idx_hbm, out_hbm, idx_v, in0, in1, in2, in3,
                  out0, out1, si0, si1, si2, si3, so0, so1):
    wid = lax.axis_index("s") * _NC + lax.axis_index("c")
    c0 = wid * _CPW

    ins, isems = (in0, in1, in2, in3), (si0, si1, si2, si3)
    outs, osems = (out0, out1), (so0, so1)

    def cp_in(s, buf, sem):
        cl, b = s // _B, s % _B
        return pltpu.make_async_copy(x_hbm.at[b, c0 + cl], buf, sem)

    def cp_out(s, buf, sem):
        cl, b = s // _B, s % _B
        return pltpu.make_async_copy(buf, out_hbm.at[b, c0 + cl], sem)

    for k in range(_DI):
        cp_in(k, ins[k], isems[k]).start()
    pltpu.sync_copy(idx_hbm.at[pl.ds(c0 * _P, _CPW * _P)], idx_v)

    def quad_body(i, carry):
        s0 = i * _DI
        for k in range(_DI):
            s = s0 + k
            pi, po = k, k % 2
            cp_in(s, ins[pi], isems[pi]).wait()

            if k < 2:
                @pl.when(i > 0)
                def _():
                    cp_out(s - 2, outs[po], osems[po]).wait()
            else:
                cp_out(s - 2, outs[po], osems[po]).wait()

            _gather_slab(ins[pi], outs[po], idx_v, s // _B)
            cp_out(s, outs[po], osems[po]).start()

            @pl.when(s + _DI < _NSLAB)
            def _():
                cp_in(s + _DI, ins[pi], isems[pi]).start()
        return carry

    lax.fori_loop(0, _NSLAB // _DI, quad_body, 0)
    cp_out(_NSLAB - 2, outs[0], osems[0]).wait()
    cp_out(_NSLAB - 1, outs[1], osems[1]).wait()


@jax.jit
def _shuffle(X, idxmat):
    kern = pl.kernel(
        _shuffle_body,
        mesh=plsc.VectorSubcoreMesh(core_axis_name="c", subcore_axis_name="s"),
        compiler_params=pltpu.CompilerParams(needs_layout_passes=False),
        out_type=jax.ShapeDtypeStruct((_B, _C, _E, _P), jnp.float32),
        scratch_types=[
            pltpu.VMEM((_CPW * _P,), jnp.int32),
            pltpu.VMEM((_E, _P), jnp.float32),
            pltpu.VMEM((_E, _P), jnp.float32),
            pltpu.VMEM((_E, _P), jnp.float32),
            pltpu.VMEM((_E, _P), jnp.float32),
            pltpu.VMEM((_E, _P), jnp.float32),
            pltpu.VMEM((_E, _P), jnp.float32),
            pltpu.SemaphoreType.DMA,
            pltpu.SemaphoreType.DMA,
            pltpu.SemaphoreType.DMA,
            pltpu.SemaphoreType.DMA,
            pltpu.SemaphoreType.DMA,
            pltpu.SemaphoreType.DMA,
        ],
    )
    return kern(X, idxmat.reshape(_C * _P))


def kernel(X, perm_tensor, random_idx):
    idx = lax.dynamic_index_in_dim(perm_tensor, random_idx, 0, keepdims=False)
    return _shuffle(X, idx)


# speedup vs baseline: 1.0238x; 1.0238x over previous
import jax
import jax.numpy as jnp
from jax import lax
from jax.experimental import pallas as pl
from jax.experimental.pallas import tpu as pltpu
from jax.experimental.pallas import tpu_sc as plsc

_B, _C, _E, _P = 32, 64, 64, 256
_L = 16
_NC, _NS = 2, 16
_NW = _NC * _NS
_CPW = _C // _NW
_NSLAB = _B * _CPW


def _gather_slab(in_ref, out_ref, idx_v, cl):
    base = [idx_v[pl.ds(cl * _P + j * _L, _L)] for j in range(_P // _L)]

    @plsc.parallel_loop(0, _E, unroll=2)
    def e_body(e):
        e_vec = jnp.full((_L,), e, dtype=jnp.int32)
        for j in range(_P // _L):
            g = plsc.load_gather(in_ref, [e_vec, base[j]])
            out_ref[e, pl.ds(j * _L, _L)] = g


_DI = 4


def _shuffle_body(x_hbm, idx_hbm, out_hbm, idx_v, in0, in1, in2, in3,
                  out0, out1, si0, si1, si2, si3, so0, so1):
    wid = lax.axis_index("s") * _NC + lax.axis_index("c")
    c0 = wid * _CPW

    ins, isems = (in0, in1, in2, in3), (si0, si1, si2, si3)
    outs, osems = (out0, out1), (so0, so1)

    def cp_in(s, buf, sem):
        cl, b = s % _CPW, s // _CPW
        return pltpu.make_async_copy(x_hbm.at[b, c0 + cl], buf, sem)

    def cp_out(s, buf, sem):
        cl, b = s % _CPW, s // _CPW
        return pltpu.make_async_copy(buf, out_hbm.at[b, c0 + cl], sem)

    for k in range(_DI):
        cp_in(k, ins[k], isems[k]).start()
    pltpu.sync_copy(idx_hbm.at[pl.ds(c0 * _P, _CPW * _P)], idx_v)

    def quad_body(i, carry):
        s0 = i * _DI
        for k in range(_DI):
            s = s0 + k
            pi, po = k, k % 2
            cp_in(s, ins[pi], isems[pi]).wait()

            if k < 2:
                @pl.when(i > 0)
                def _():
                    cp_out(s - 2, outs[po], osems[po]).wait()
            else:
                cp_out(s - 2, outs[po], osems[po]).wait()

            _gather_slab(ins[pi], outs[po], idx_v, k % _CPW)
            cp_out(s, outs[po], osems[po]).start()

            @pl.when(s + _DI < _NSLAB)
            def _():
                cp_in(s + _DI, ins[pi], isems[pi]).start()
        return carry

    lax.fori_loop(0, _NSLAB // _DI, quad_body, 0)
    cp_out(_NSLAB - 2, outs[0], osems[0]).wait()
    cp_out(_NSLAB - 1, outs[1], osems[1]).wait()


@jax.jit
def _shuffle(X, idxmat):
    kern = pl.kernel(
        _shuffle_body,
        mesh=plsc.VectorSubcoreMesh(core_axis_name="c", subcore_axis_name="s"),
        compiler_params=pltpu.CompilerParams(needs_layout_passes=False),
        out_type=jax.ShapeDtypeStruct((_B, _C, _E, _P), jnp.float32),
        scratch_types=[
            pltpu.VMEM((_CPW * _P,), jnp.int32),
            pltpu.VMEM((_E, _P), jnp.float32),
            pltpu.VMEM((_E, _P), jnp.float32),
            pltpu.VMEM((_E, _P), jnp.float32),
            pltpu.VMEM((_E, _P), jnp.float32),
            pltpu.VMEM((_E, _P), jnp.float32),
            pltpu.VMEM((_E, _P), jnp.float32),
            pltpu.SemaphoreType.DMA,
            pltpu.SemaphoreType.DMA,
            pltpu.SemaphoreType.DMA,
            pltpu.SemaphoreType.DMA,
            pltpu.SemaphoreType.DMA,
            pltpu.SemaphoreType.DMA,
        ],
    )
    return kern(X, idxmat.reshape(_C * _P))


def kernel(X, perm_tensor, random_idx):
    idx = lax.dynamic_index_in_dim(perm_tensor, random_idx, 0, keepdims=False)
    return _shuffle(X, idx)
